# Initial kernel scaffold; baseline (speedup 1.0000x reference)
import jax
import jax.numpy as jnp
from jax.experimental import pallas as pl


def kernel(input_coords_cpu, num_atoms_of_type_cpu, offsets_cpu):
    raise NotImplementedError("write your pallas kernel here")


# speedup vs baseline: 21.3253x; 21.3253x over previous
import functools

import numpy as np
import jax
import jax.numpy as jnp
from jax import lax
from jax.experimental import pallas as pl
from jax.experimental.pallas import tpu as pltpu
from jax.experimental.pallas import tpu_sc as plsc

_BOX = 120
_T = 11
_B = 2
_A = 2200
_NCH = _B * _T
_SLABS = 16
_ROWS = _BOX * _BOX
_SLAB_ROWS = _ROWS // _SLABS
_SLAB_WORDS = _SLAB_ROWS * _BOX
_NTILES = 32
_ROUNDS = (_NCH * _SLABS) // _NTILES

_lane = np.arange(128)
_DX = np.where(_lane < 125, _lane // 25 - 2, 0).astype(np.int32)
_DY = np.where(_lane < 125, (_lane // 5) % 5 - 2, 0).astype(np.int32)
_DZ = np.where(_lane < 125, _lane % 5 - 2, 0).astype(np.int32)
_LM = (_lane < 125).astype(np.int32)


def _sc_body(coords_hbm, lo_hbm, num_hbm, cb_hbm, dx_hbm, dy_hbm, dz_hbm,
             lm_hbm, out_hbm, coords_v, lov, numv, cbv, dxv, dyv, dzv, lmv,
             acc):
    cid = lax.axis_index("c")
    sid = lax.axis_index("s")
    wid = sid * 2 + cid

    pltpu.sync_copy(coords_hbm, coords_v)
    pltpu.sync_copy(lo_hbm, lov)
    pltpu.sync_copy(num_hbm, numv)
    pltpu.sync_copy(cb_hbm, cbv)
    pltpu.sync_copy(dx_hbm, dxv)
    pltpu.sync_copy(dy_hbm, dyv)
    pltpu.sync_copy(dz_hbm, dzv)
    pltpu.sync_copy(lm_hbm, lmv)

    for r in range(_ROUNDS):
        tau = r * _NTILES + wid
        ch = tau // _SLABS
        slab = tau % _SLABS
        rowbase = slab * _SLAB_ROWS

        def _zero(i, carry):
            acc[pl.ds(i * 16, 16)] = jnp.zeros((16,), jnp.float32)
            return carry
        lax.fori_loop(0, _SLAB_WORDS // 16, _zero, 0)

        chv = jnp.broadcast_to(ch, (16,))
        lo_v = plsc.load_gather(lov, [chv])
        nm_v = plsc.load_gather(numv, [chv])
        cb_v = plsc.load_gather(cbv, [chv])
        hi_v = lo_v + nm_v

        def _splat(i, carry):
            av = lo_v + i
            mv = av < hi_v
            basev = jnp.where(mv, cb_v + 3 * av, 0)
            xv = plsc.load_gather(coords_v, [basev])
            yv = plsc.load_gather(coords_v, [basev + 1])
            zv = plsc.load_gather(coords_v, [basev + 2])
            x0 = xv.astype(jnp.int32)
            y0 = yv.astype(jnp.int32)
            z0 = zv.astype(jnp.int32)
            for j in range(8):
                cx = x0 + dxv[pl.ds(j * 16, 16)]
                cy = y0 + dyv[pl.ds(j * 16, 16)]
                cz = z0 + dzv[pl.ds(j * 16, 16)]
                fx = cx.astype(jnp.float32) - xv
                fy = cy.astype(jnp.float32) - yv
                fz = cz.astype(jnp.float32) - zv
                val = jnp.exp(-(fx * fx + fy * fy + fz * fz))
                row = cx * _BOX + cy - rowbase
                m = (mv & (row >= 0) & (row < _SLAB_ROWS)
                     & (lmv[pl.ds(j * 16, 16)] > 0))
                lidx = jnp.where(m, row * _BOX + cz, 0)
                plsc.addupdate_scatter(acc, [lidx], val, mask=m)
            return carry

        lax.fori_loop(0, 200, _splat, 0)

        pltpu.sync_copy(acc, out_hbm.at[tau])


@jax.jit
def _typed_splat(coords_flat, lo32, num32, cb32):
    mesh = plsc.VectorSubcoreMesh(core_axis_name="c", subcore_axis_name="s")
    f = functools.partial(
        pl.kernel,
        out_type=jax.ShapeDtypeStruct((_NCH * _SLABS, _SLAB_WORDS),
                                      jnp.float32),
        mesh=mesh,
        compiler_params=pltpu.CompilerParams(needs_layout_passes=False),
        scratch_types=[
            pltpu.VMEM((3 * _A * _B,), jnp.float32),
            pltpu.VMEM((32,), jnp.int32),
            pltpu.VMEM((32,), jnp.int32),
            pltpu.VMEM((32,), jnp.int32),
            pltpu.VMEM((128,), jnp.int32),
            pltpu.VMEM((128,), jnp.int32),
            pltpu.VMEM((128,), jnp.int32),
            pltpu.VMEM((128,), jnp.int32),
            pltpu.VMEM((_SLAB_WORDS,), jnp.float32),
        ],
    )(_sc_body)
    return f(coords_flat, lo32, num32, cb32,
             jnp.asarray(_DX), jnp.asarray(_DY), jnp.asarray(_DZ),
             jnp.asarray(_LM))


def kernel(input_coords_cpu, num_atoms_of_type_cpu, offsets_cpu):
    coords_flat = input_coords_cpu.reshape(-1)
    b_idx = jnp.arange(_NCH, dtype=jnp.int32) // _T
    t_idx = jnp.arange(_NCH, dtype=jnp.int32) % _T
    lo = offsets_cpu[b_idx, t_idx].astype(jnp.int32)
    nm = num_atoms_of_type_cpu[b_idx, t_idx].astype(jnp.int32)
    cb = (b_idx * (3 * _A)).astype(jnp.int32)
    z = jnp.zeros((32,), jnp.int32)
    lo32 = z.at[:_NCH].set(lo)
    num32 = z.at[:_NCH].set(nm)
    cb32 = z.at[:_NCH].set(cb)
    out = _typed_splat(coords_flat, lo32, num32, cb32)
    return out.reshape(_B, _T, _BOX, _BOX, _BOX)


# speedup vs baseline: 73.7541x; 3.4585x over previous
import functools

import numpy as np
import jax
import jax.numpy as jnp
from jax import lax
from jax.experimental import pallas as pl
from jax.experimental.pallas import tpu as pltpu
from jax.experimental.pallas import tpu_sc as plsc

_BOX = 120
_T = 11
_B = 2
_A = 2200
_NCH = _B * _T
_SLABS = 16
_ROWS = _BOX * _BOX
_SLAB_ROWS = _ROWS // _SLABS
_SLAB_WORDS = _SLAB_ROWS * _BOX
_NTILES = 32
_ROUNDS = (_NCH * _SLABS) // _NTILES
_MAXN = 224

_lane = np.arange(128)
_dxi = _lane // 25 - 2
_dyi = (_lane // 5) % 5 - 2
_dzi = _lane % 5 - 2
_DXF = np.where(_lane < 125, _dxi, 0).astype(np.float32)
_DYF = np.where(_lane < 125, _dyi, 0).astype(np.float32)
_DZF = np.where(_lane < 125, _dzi, 0).astype(np.float32)
_DROW = np.where(_lane < 125, _dxi * _BOX + _dyi, -10**6).astype(np.int32)
_DCELL = np.where(_lane < 125, (_dxi * _BOX + _dyi) * _BOX + _dzi,
                  -10**6).astype(np.int32)


def _sc_body(coords_hbm, lo_hbm, num_hbm, cb_hbm, dxf_hbm, dyf_hbm, dzf_hbm,
             drow_hbm, dcell_hbm, out_hbm,
             coords_v, lov, numv, cbv, dxfv, dyfv, dzfv, drowv, dcellv,
             surv, cellcs, acc):
    cid = lax.axis_index("c")
    sid = lax.axis_index("s")
    wid = sid * 2 + cid

    pltpu.sync_copy(coords_hbm, coords_v)
    pltpu.sync_copy(lo_hbm, lov)
    pltpu.sync_copy(num_hbm, numv)
    pltpu.sync_copy(cb_hbm, cbv)
    pltpu.sync_copy(dxf_hbm, dxfv)
    pltpu.sync_copy(dyf_hbm, dyfv)
    pltpu.sync_copy(dzf_hbm, dzfv)
    pltpu.sync_copy(drow_hbm, drowv)
    pltpu.sync_copy(dcell_hbm, dcellv)

    iota = lax.iota(jnp.int32, 16)
    zero16 = jnp.zeros((16,), jnp.float32)

    def _zero(i, carry):
        acc[pl.ds(i * 16, 16)] = zero16
        return carry
    lax.fori_loop(0, _SLAB_WORDS // 16, _zero, 0)

    for r in range(_ROUNDS):
        tau = r * _NTILES + wid
        ch = tau // _SLABS
        slab = tau % _SLABS
        rowbase = slab * _SLAB_ROWS

        chv = jnp.broadcast_to(ch, (16,))
        lo_v = plsc.load_gather(lov, [chv])
        nm_v = plsc.load_gather(numv, [chv])
        cb_v = plsc.load_gather(cbv, [chv])
        hi_v = lo_v + nm_v
        n_s = jnp.max(nm_v)

        def _fcond(c):
            return c[0] * 16 < n_s

        def _fbody(c):
            i, cnt = c
            av = lo_v + i * 16 + iota
            mv = av < hi_v
            basev = jnp.where(mv, cb_v + 3 * av, 0)
            xv = plsc.load_gather(coords_v, [basev])
            yv = plsc.load_gather(coords_v, [basev + 1])
            x0 = xv.astype(jnp.int32)
            y0 = yv.astype(jnp.int32)
            rmin = (x0 - 2) * _BOX + (y0 - 2)
            rmax = (x0 + 2) * _BOX + (y0 + 2)
            keep = mv & (rmax >= rowbase) & (rmin < rowbase + _SLAB_ROWS)
            plsc.store_compressed(surv.at[pl.ds(cnt, 16)], av, mask=keep)
            return i + 1, cnt + jnp.sum(keep.astype(jnp.int32))

        _, cnt_s = lax.while_loop(_fcond, _fbody,
                                  (jnp.int32(0), jnp.int32(0)))

        def _scond(j):
            return j < cnt_s

        def _sbody(j):
            jv = jnp.broadcast_to(j, (16,))
            aidv = plsc.load_gather(surv, [jv])
            basev = cb_v + 3 * aidv
            xv = plsc.load_gather(coords_v, [basev])
            yv = plsc.load_gather(coords_v, [basev + 1])
            zv = plsc.load_gather(coords_v, [basev + 2])
            x0 = xv.astype(jnp.int32)
            y0 = yv.astype(jnp.int32)
            z0 = zv.astype(jnp.int32)
            fracx = xv - x0.astype(jnp.float32)
            fracy = yv - y0.astype(jnp.float32)
            fracz = zv - z0.astype(jnp.float32)
            rowc = x0 * _BOX + y0 - rowbase
            cellc = rowc * _BOX + z0
            cellcs[pl.ds(j * 16, 16)] = cellc
            for g in range(8):
                sl = pl.ds(g * 16, 16)
                fx = dxfv[sl] - fracx
                fy = dyfv[sl] - fracy
                fz = dzfv[sl] - fracz
                val = jnp.exp(-(fx * fx + fy * fy + fz * fz))
                row = rowc + drowv[sl]
                m = (row >= 0) & (row < _SLAB_ROWS)
                lidx = cellc + dcellv[sl]
                plsc.addupdate_scatter(acc, [lidx], val, mask=m)
            return j + 1

        lax.while_loop(_scond, _sbody, jnp.int32(0))

        pltpu.sync_copy(acc, out_hbm.at[tau])

        if r < _ROUNDS - 1:
            def _zbody(j):
                cellc = cellcs[pl.ds(j * 16, 16)]
                for g in range(8):
                    lidx = cellc + dcellv[pl.ds(g * 16, 16)]
                    lidx = jnp.clip(lidx, 0, _SLAB_WORDS - 1)
                    plsc.store_scatter(acc, [lidx], zero16)
                return j + 1

            lax.while_loop(_scond, _zbody, jnp.int32(0))


@jax.jit
def _typed_splat(coords_flat, lo32, num32, cb32):
    mesh = plsc.VectorSubcoreMesh(core_axis_name="c", subcore_axis_name="s")
    f = functools.partial(
        pl.kernel,
        out_type=jax.ShapeDtypeStruct((_NCH * _SLABS, _SLAB_WORDS),
                                      jnp.float32),
        mesh=mesh,
        compiler_params=pltpu.CompilerParams(needs_layout_passes=False),
        scratch_types=[
            pltpu.VMEM((3 * _A * _B,), jnp.float32),
            pltpu.VMEM((32,), jnp.int32),
            pltpu.VMEM((32,), jnp.int32),
            pltpu.VMEM((32,), jnp.int32),
            pltpu.VMEM((128,), jnp.float32),
            pltpu.VMEM((128,), jnp.float32),
            pltpu.VMEM((128,), jnp.float32),
            pltpu.VMEM((128,), jnp.int32),
            pltpu.VMEM((128,), jnp.int32),
            pltpu.VMEM((_MAXN,), jnp.int32),
            pltpu.VMEM((_MAXN * 16,), jnp.int32),
            pltpu.VMEM((_SLAB_WORDS,), jnp.float32),
        ],
    )(_sc_body)
    return f(coords_flat, lo32, num32, cb32,
             jnp.asarray(_DXF), jnp.asarray(_DYF), jnp.asarray(_DZF),
             jnp.asarray(_DROW), jnp.asarray(_DCELL))


def kernel(input_coords_cpu, num_atoms_of_type_cpu, offsets_cpu):
    coords_flat = input_coords_cpu.reshape(-1)
    b_idx = jnp.arange(_NCH, dtype=jnp.int32) // _T
    t_idx = jnp.arange(_NCH, dtype=jnp.int32) % _T
    lo = offsets_cpu[b_idx, t_idx].astype(jnp.int32)
    nm = num_atoms_of_type_cpu[b_idx, t_idx].astype(jnp.int32)
    cb = (b_idx * (3 * _A)).astype(jnp.int32)
    z = jnp.zeros((32,), jnp.int32)
    lo32 = z.at[:_NCH].set(lo)
    num32 = z.at[:_NCH].set(nm)
    cb32 = z.at[:_NCH].set(cb)
    out = _typed_splat(coords_flat, lo32, num32, cb32)
    return out.reshape(_B, _T, _BOX, _BOX, _BOX)


# speedup vs baseline: 150.7417x; 2.0438x over previous
import functools

import numpy as np
import jax
import jax.numpy as jnp
from jax import lax
from jax.experimental import pallas as pl
from jax.experimental.pallas import tpu as pltpu
from jax.experimental.pallas import tpu_sc as plsc

_BOX = 120
_T = 11
_B = 2
_A = 2200
_NCH = _B * _T
_FACE = _BOX * _BOX
_NXMAX = 8
_MAXN = 224

_lane = np.arange(128)
_dxi = _lane // 25 - 2
_dyi = (_lane // 5) % 5 - 2
_dzi = _lane % 5 - 2
_TABF = np.concatenate([
    np.where(_lane < 125, _dxi, 0),
    np.where(_lane < 125, _dyi, 0),
    np.where(_lane < 125, _dzi, 0),
]).astype(np.float32)
_TABI = np.concatenate([
    np.where(_lane < 125, _dxi, -10**6),
    np.where(_lane < 125, _dyi, 0),
    np.where(_lane < 125, _dzi, 0),
]).astype(np.int32)


def _sc_body(coords_hbm, lo_hbm, num_hbm, tabf_hbm, tabi_hbm, zeros_hbm,
             out_hbm, coords_v, lov, numv, tabf, tabi, surv, acc):
    cid = lax.axis_index("c")
    sid = lax.axis_index("s")
    wid = sid * 2 + cid
    bat = wid % 2
    k = wid // 2
    nx = jnp.where(k < 8, 8, 7)
    xbase = jnp.where(k < 8, 8 * k, 7 * k + 8)

    pltpu.sync_copy(coords_hbm.at[bat], coords_v)
    pltpu.sync_copy(lo_hbm, lov)
    pltpu.sync_copy(num_hbm, numv)
    pltpu.sync_copy(tabf_hbm, tabf)
    pltpu.sync_copy(tabi_hbm, tabi)
    pltpu.sync_copy(zeros_hbm, acc)

    iota = lax.iota(jnp.int32, 16)
    zero16 = jnp.zeros((16,), jnp.float32)
    nxv = jnp.broadcast_to(nx, (16,))
    xbasev = jnp.broadcast_to(xbase, (16,))

    for r in range(_T):
        ch = bat * _T + r

        chv = jnp.broadcast_to(ch, (16,))
        lo_v = plsc.load_gather(lov, [chv])
        nm_v = plsc.load_gather(numv, [chv])
        hi_v = lo_v + nm_v
        n_s = jnp.max(nm_v)

        def _fcond(c):
            return c[0] * 16 < n_s

        def _fbody(c):
            i, cnt = c
            av = lo_v + i * 16 + iota
            mv = av < hi_v
            basev = jnp.where(mv, 3 * av, 0)
            xv = plsc.load_gather(coords_v, [basev])
            x0 = xv.astype(jnp.int32)
            keep = mv & (x0 + 2 >= xbasev) & (x0 - 2 < xbasev + nxv)
            plsc.store_compressed(surv.at[pl.ds(cnt, 16)], av, mask=keep)
            return i + 1, cnt + jnp.sum(keep.astype(jnp.int32))

        _, cnt_s = lax.while_loop(_fcond, _fbody,
                                  (jnp.int32(0), jnp.int32(0)))

        def _scond(j):
            return j < cnt_s

        def _sbody(j):
            jv = jnp.broadcast_to(j, (16,))
            aidv = plsc.load_gather(surv, [jv])
            basev = 3 * aidv
            xv = plsc.load_gather(coords_v, [basev])
            yv = plsc.load_gather(coords_v, [basev + 1])
            zv = plsc.load_gather(coords_v, [basev + 2])
            x0 = xv.astype(jnp.int32)
            y0 = yv.astype(jnp.int32)
            z0 = zv.astype(jnp.int32)
            fracx = xv - x0.astype(jnp.float32)
            fracy = yv - y0.astype(jnp.float32)
            fracz = zv - z0.astype(jnp.float32)
            xrel = x0 - xbasev
            for g in range(8):
                fx = tabf[pl.ds(g * 16, 16)] - fracx
                fy = tabf[pl.ds(128 + g * 16, 16)] - fracy
                fz = tabf[pl.ds(256 + g * 16, 16)] - fracz
                val = jnp.exp(-(fx * fx + fy * fy + fz * fz))
                ix = xrel + tabi[pl.ds(g * 16, 16)]
                iy = y0 + tabi[pl.ds(128 + g * 16, 16)]
                iz = z0 + tabi[pl.ds(256 + g * 16, 16)]
                m = (ix >= 0) & (ix < nxv)
                plsc.addupdate_scatter(acc, [ix, iy, iz], val, mask=m)
            return j + 1

        lax.while_loop(_scond, _sbody, jnp.int32(0))

        pltpu.sync_copy(acc.at[pl.ds(0, 7)],
                        out_hbm.at[bat, r, pl.ds(xbase, 7)])

        @pl.when(k < 8)
        def _():
            pltpu.sync_copy(acc.at[7], out_hbm.at[bat, r, xbase + 7])

        if r < _T - 1:
            def _zbody(j):
                jv = jnp.broadcast_to(j, (16,))
                aidv = plsc.load_gather(surv, [jv])
                basev = 3 * aidv
                x0 = plsc.load_gather(coords_v, [basev]).astype(jnp.int32)
                y0 = plsc.load_gather(coords_v, [basev + 1]).astype(jnp.int32)
                z0 = plsc.load_gather(coords_v, [basev + 2]).astype(jnp.int32)
                xrel = x0 - xbasev
                for g in range(8):
                    ix = xrel + tabi[pl.ds(g * 16, 16)]
                    ix = jnp.clip(ix, 0, _NXMAX - 1)
                    iy = y0 + tabi[pl.ds(128 + g * 16, 16)]
                    iz = z0 + tabi[pl.ds(256 + g * 16, 16)]
                    plsc.store_scatter(acc, [ix, iy, iz], zero16)
                return j + 1

            lax.while_loop(_scond, _zbody, jnp.int32(0))


@jax.jit
def _typed_splat(coords2d, lo32, num32):
    mesh = plsc.VectorSubcoreMesh(core_axis_name="c", subcore_axis_name="s")
    f = functools.partial(
        pl.kernel,
        out_type=jax.ShapeDtypeStruct((_B, _T, _BOX, _BOX, _BOX),
                                      jnp.float32),
        mesh=mesh,
        compiler_params=pltpu.CompilerParams(needs_layout_passes=False),
        scratch_types=[
            pltpu.VMEM((3 * _A,), jnp.float32),
            pltpu.VMEM((32,), jnp.int32),
            pltpu.VMEM((32,), jnp.int32),
            pltpu.VMEM((384,), jnp.float32),
            pltpu.VMEM((384,), jnp.int32),
            pltpu.VMEM((_MAXN,), jnp.int32),
            pltpu.VMEM((_NXMAX, _BOX, _BOX), jnp.float32),
        ],
    )(_sc_body)
    return f(coords2d, lo32, num32, jnp.asarray(_TABF), jnp.asarray(_TABI),
             jnp.zeros((_NXMAX, _BOX, _BOX), jnp.float32))


def kernel(input_coords_cpu, num_atoms_of_type_cpu, offsets_cpu):
    z = jnp.zeros((32,), jnp.int32)
    lo32 = z.at[:_NCH].set(offsets_cpu.astype(jnp.int32).reshape(-1))
    num32 = z.at[:_NCH].set(num_atoms_of_type_cpu.astype(jnp.int32).reshape(-1))
    return _typed_splat(input_coords_cpu, lo32, num32)


# speedup vs baseline: 151.3109x; 1.0038x over previous
import functools

import numpy as np
import jax
import jax.numpy as jnp
from jax import lax
from jax.experimental import pallas as pl
from jax.experimental.pallas import tpu as pltpu
from jax.experimental.pallas import tpu_sc as plsc

_BOX = 120
_T = 11
_B = 2
_A = 2200
_NCH = _B * _T
_FACE = _BOX * _BOX
_NXMAX = 8
_MAXN = 224

_lane = np.arange(128)
_dxi = _lane // 25 - 2
_dyi = (_lane // 5) % 5 - 2
_dzi = _lane % 5 - 2
_TABF = np.concatenate([
    np.where(_lane < 125, _dxi, 0),
    np.where(_lane < 125, _dyi, 0),
    np.where(_lane < 125, _dzi, 0),
]).astype(np.float32)
_TABI = np.concatenate([
    np.where(_lane < 125, _dxi, -10**6),
    np.where(_lane < 125, _dyi, 0),
    np.where(_lane < 125, _dzi, 0),
]).astype(np.int32)


def _sc_body(coords_hbm, lo_hbm, num_hbm, tabf_hbm, tabi_hbm, zeros_hbm,
             out_hbm, coords_v, lov, numv, tabf, tabi, surv, acc):
    cid = lax.axis_index("c")
    sid = lax.axis_index("s")
    wid = sid * 2 + cid
    bat = wid % 2
    k = wid // 2
    nx = jnp.where(k < 8, 8, 7)
    xbase = jnp.where(k < 8, 8 * k, 7 * k + 8)

    pltpu.sync_copy(coords_hbm.at[bat], coords_v)
    pltpu.sync_copy(lo_hbm, lov)
    pltpu.sync_copy(num_hbm, numv)
    pltpu.sync_copy(tabf_hbm, tabf)
    pltpu.sync_copy(tabi_hbm, tabi)
    pltpu.sync_copy(zeros_hbm, acc)

    iota = lax.iota(jnp.int32, 16)
    zero16 = jnp.zeros((16,), jnp.float32)
    nxv = jnp.broadcast_to(nx, (16,))
    xbasev = jnp.broadcast_to(xbase, (16,))

    for r in range(_T):
        chv = jnp.broadcast_to(bat * _T + r, (16,))
        lo_v = plsc.load_gather(lov, [chv])
        nm_v = plsc.load_gather(numv, [chv])
        hi_v = lo_v + nm_v
        n_s = jnp.max(nm_v)

        def _fcond(c):
            return c[0] * 16 < n_s

        def _fbody(c):
            i, cnt = c
            av = lo_v + i * 16 + iota
            mv = av < hi_v
            basev = jnp.where(mv, 3 * av, 0)
            xv = plsc.load_gather(coords_v, [basev])
            x0 = xv.astype(jnp.int32)
            keep = mv & (x0 + 2 >= xbasev) & (x0 - 2 < xbasev + nxv)
            plsc.store_compressed(surv.at[pl.ds(cnt, 16)], av, mask=keep)
            return i + 1, cnt + jnp.sum(keep.astype(jnp.int32))

        _, cnt_s = lax.while_loop(_fcond, _fbody,
                                  (jnp.int32(0), jnp.int32(0)))

        def _scond(j):
            return j < cnt_s

        def _sbody(j):
            jv = jnp.broadcast_to(j, (16,))
            aidv = plsc.load_gather(surv, [jv])
            basev = 3 * aidv
            xv = plsc.load_gather(coords_v, [basev])
            yv = plsc.load_gather(coords_v, [basev + 1])
            zv = plsc.load_gather(coords_v, [basev + 2])
            x0 = xv.astype(jnp.int32)
            y0 = yv.astype(jnp.int32)
            z0 = zv.astype(jnp.int32)
            fracx = xv - x0.astype(jnp.float32)
            fracy = yv - y0.astype(jnp.float32)
            fracz = zv - z0.astype(jnp.float32)
            xrel = x0 - xbasev
            for g in range(8):
                fx = tabf[pl.ds(g * 16, 16)] - fracx
                fy = tabf[pl.ds(128 + g * 16, 16)] - fracy
                fz = tabf[pl.ds(256 + g * 16, 16)] - fracz
                val = jnp.exp(-(fx * fx + fy * fy + fz * fz))
                ix = xrel + tabi[pl.ds(g * 16, 16)]
                iy = y0 + tabi[pl.ds(128 + g * 16, 16)]
                iz = z0 + tabi[pl.ds(256 + g * 16, 16)]
                m = (ix >= 0) & (ix < nxv)
                plsc.addupdate_scatter(acc, [ix, iy, iz], val, mask=m)
            return j + 1

        lax.while_loop(_scond, _sbody, jnp.int32(0))

        pltpu.sync_copy(acc.at[pl.ds(0, 7)],
                        out_hbm.at[bat, r, pl.ds(xbase, 7)])

        @pl.when(k < 8)
        def _():
            pltpu.sync_copy(acc.at[7], out_hbm.at[bat, r, xbase + 7])

        if r < _T - 1:
            def _zbody(j):
                jv = jnp.broadcast_to(j, (16,))
                aidv = plsc.load_gather(surv, [jv])
                basev = 3 * aidv
                x0 = plsc.load_gather(coords_v, [basev]).astype(jnp.int32)
                y0 = plsc.load_gather(coords_v, [basev + 1]).astype(jnp.int32)
                z0 = plsc.load_gather(coords_v, [basev + 2]).astype(jnp.int32)
                xrel = x0 - xbasev
                for g in range(8):
                    ix = xrel + tabi[pl.ds(g * 16, 16)]
                    ix = jnp.clip(ix, 0, _NXMAX - 1)
                    iy = y0 + tabi[pl.ds(128 + g * 16, 16)]
                    iz = z0 + tabi[pl.ds(256 + g * 16, 16)]
                    plsc.store_scatter(acc, [ix, iy, iz], zero16)
                return j + 1

            lax.while_loop(_scond, _zbody, jnp.int32(0))


@jax.jit
def _typed_splat(coords2d, lo32, num32):
    mesh = plsc.VectorSubcoreMesh(core_axis_name="c", subcore_axis_name="s")
    f = functools.partial(
        pl.kernel,
        out_type=jax.ShapeDtypeStruct((_B, _T, _BOX, _BOX, _BOX),
                                      jnp.float32),
        mesh=mesh,
        compiler_params=pltpu.CompilerParams(needs_layout_passes=False),
        scratch_types=[
            pltpu.VMEM((3 * _A,), jnp.float32),
            pltpu.VMEM((_NCH,), jnp.int32),
            pltpu.VMEM((_NCH,), jnp.int32),
            pltpu.VMEM((384,), jnp.float32),
            pltpu.VMEM((384,), jnp.int32),
            pltpu.VMEM((_MAXN,), jnp.int32),
            pltpu.VMEM((_NXMAX, _BOX, _BOX), jnp.float32),
        ],
    )(_sc_body)
    return f(coords2d, lo32, num32, jnp.asarray(_TABF), jnp.asarray(_TABI),
             jnp.zeros((_NXMAX, _BOX, _BOX), jnp.float32))


def kernel(input_coords_cpu, num_atoms_of_type_cpu, offsets_cpu):
    lo32 = offsets_cpu.astype(jnp.int32).reshape(-1)
    num32 = num_atoms_of_type_cpu.astype(jnp.int32).reshape(-1)
    return _typed_splat(input_coords_cpu, lo32, num32)
